# baseline (speedup 1.0000x reference)
import functools

import jax
import jax.numpy as jnp
from jax import lax
from jax.experimental import pallas as pl
from jax.experimental.pallas import tpu as pltpu
from jax.experimental.pallas import tpu_sc as plsc


def _body(d0_ref, d1_ref, wt_ref, bp_ref, wm_ref, bm_ref,
          kn_ref, ls_ref, kaw_ref, er1_ref, er2_ref, ci1_ref, ci2_ref,
          md1t_s, S_s, ra_s, l0m_s, ls1_s, l1m_s,
          cmax_s, csum_s, cc_s, rv1_s, rv2_s, ri1_s, ri2_s,
          cv1_s, cv2_s, ci1_s, ci2_s,
          *, BM, n, d, nmb, s):
    mi = pl.program_id(1)
    NEG = jnp.float32(-jnp.inf)

    @pl.when(mi == 0)
    def _init():
        md1 = (jnp.dot(d1_ref[...], wt_ref[...],
                       preferred_element_type=jnp.float32) + bp_ref[...]) / s
        md1t_s[...] = md1.T
        z1 = jnp.dot(d1_ref[...], wm_ref[...],
                     preferred_element_type=jnp.float32) + bm_ref[...]
        ls1_s[...] = jax.nn.log_sigmoid(z1).T
        l1m_s[...] = jax.nn.log_sigmoid(-z1).T
        cmax_s[...] = jnp.full((1, n), NEG, jnp.float32)
        csum_s[...] = jnp.zeros((1, n), jnp.float32)

    @pl.when(mi < nmb)
    def _phase_a():
        d0 = d0_ref[...]
        md0 = (jnp.dot(d0, wt_ref[...],
                       preferred_element_type=jnp.float32) + bp_ref[...]) / s
        kn = jnp.dot(md0, md1t_s[...], preferred_element_type=jnp.float32)
        kn_ref[...] = kn
        S_s[pl.ds(mi * BM, BM), :] = kn
        rmax = jnp.max(kn, axis=1, keepdims=True)
        rlse = rmax + jnp.log(
            jnp.sum(jnp.exp(kn - rmax), axis=1, keepdims=True))
        z0 = jnp.dot(d0, wm_ref[...],
                     preferred_element_type=jnp.float32) + bm_ref[...]
        ra_s[pl.ds(mi * BM, BM), :] = jax.nn.log_sigmoid(z0) - rlse
        l0m_s[pl.ds(mi * BM, BM), :] = jax.nn.log_sigmoid(-z0)
        bmax = jnp.max(kn, axis=0, keepdims=True)
        prev = cmax_s[...]
        newm = jnp.maximum(prev, bmax)
        csum_s[...] = csum_s[...] * jnp.exp(prev - newm) + jnp.sum(
            jnp.exp(kn - newm), axis=0, keepdims=True)
        cmax_s[...] = newm

    @pl.when(mi == nmb)
    def _fold_col():
        cc_s[...] = ls1_s[...] - (cmax_s[...] + jnp.log(csum_s[...]))

    @pl.when((mi >= nmb) & (mi < 2 * nmb))
    def _phase_b():
        k = mi - nmb
        kn = S_s[pl.ds(k * BM, BM), :]
        core = (kn + kn) + ra_s[pl.ds(k * BM, BM), :] + cc_s[...]
        ls_ref[:, 0:n] = core
        ls_ref[:, n:n + 1] = l0m_s[pl.ds(k * BM, BM), :]
        kaw_ref[...] = jnp.zeros((BM, n // 4), jnp.int32)
        jj = jax.lax.broadcasted_iota(jnp.int32, (BM, n), 1)
        v1 = jnp.max(core, axis=1, keepdims=True)
        i1 = jnp.min(jnp.where(core == v1, jj, n), axis=1, keepdims=True)
        sm = jnp.where(jj == i1, NEG, core)
        v2 = jnp.max(sm, axis=1, keepdims=True)
        i2 = jnp.min(jnp.where(sm == v2, jj, n), axis=1, keepdims=True)
        rv1_s[pl.ds(k * BM, BM), :] = v1
        rv2_s[pl.ds(k * BM, BM), :] = v2
        ri1_s[pl.ds(k * BM, BM), :] = i1
        ri2_s[pl.ds(k * BM, BM), :] = i2
        ii = jax.lax.broadcasted_iota(jnp.int32, (BM, n), 0)
        bv1 = jnp.max(core, axis=0, keepdims=True)
        bl1 = jnp.min(jnp.where(core == bv1, ii, BM), axis=0, keepdims=True)
        sc = jnp.where(ii == bl1, NEG, core)
        bv2 = jnp.max(sc, axis=0, keepdims=True)
        bl2 = jnp.min(jnp.where(sc == bv2, ii, BM), axis=0, keepdims=True)
        gbi1 = bl1 + k * BM
        gbi2 = bl2 + k * BM
        first = k == 0
        pv1 = jnp.where(first, NEG, cv1_s[...])
        pi1 = jnp.where(first, 0, ci1_s[...])
        pv2 = jnp.where(first, NEG, cv2_s[...])
        pi2 = jnp.where(first, 0, ci2_s[...])
        run1 = pv1 >= bv1
        cv1_s[...] = jnp.where(run1, pv1, bv1)
        ci1_s[...] = jnp.where(run1, pi1, gbi1)
        cv2_s[...] = jnp.where(run1, jnp.maximum(pv2, bv1),
                               jnp.maximum(pv1, bv2))
        ci2_s[...] = jnp.where(run1, jnp.where(pv2 >= bv1, pi2, gbi1),
                               jnp.where(pv1 >= bv2, pi1, gbi2))

    @pl.when(mi == 2 * nmb)
    def _final():
        ls_ref[0:1, 0:n] = l1m_s[...]
        ls_ref[0:1, n:n + 1] = jnp.zeros((1, 1), jnp.float32)
        er1_ref[...] = jnp.where(jnp.exp(rv1_s[...]) > 0.0,
                                 ri1_s[...], -1).T
        er2_ref[...] = jnp.where(jnp.exp(rv2_s[...]) > 0.0,
                                 ri2_s[...], -1).T
        ci1_ref[...] = ci1_s[...]
        ci2_ref[...] = ci2_s[...]


def _sc_patch(er1, er2, ci1f, ci2f, kaw, b, m, n):
    bm_rows = er1.shape[0]
    nw4 = n // 4
    info = plsc.get_sparse_core_info()
    NC, NS, L = info.num_cores, info.num_subcores, info.num_lanes
    NW = NC * NS
    rows_pw = bm_rows // NW
    nch = rows_pw // L
    ndma = rows_pw // 128
    mesh = plsc.VectorSubcoreMesh(core_axis_name="c", subcore_axis_name="s")

    def k(er1_hbm, er2_hbm, ci1_hbm, ci2_hbm, kaw_out,
          er1_v, er2_v, cj1_v, cj2_v, g11_v, g21_v, g12_v, g22_v,
          val_v, sem, sem2):
        wid = lax.axis_index("s") * NC + lax.axis_index("c")
        base_row = wid * rows_pw
        bi = base_row // m
        i_local = base_row - bi * m
        pltpu.sync_copy(er1_hbm.at[pl.ds(base_row, rows_pw)], er1_v)
        pltpu.sync_copy(er2_hbm.at[pl.ds(base_row, rows_pw)], er2_v)
        for c in range(nch):
            off = c * L
            e1 = er1_v[pl.ds(off, L)]
            e2 = er2_v[pl.ds(off, L)]
            cj1_v[pl.ds(off, L)] = bi * n + jnp.maximum(e1, 0)
            cj2_v[pl.ds(off, L)] = bi * n + jnp.maximum(e2, 0)
        gathers = []
        for p in range(ndma):
            sl = pl.ds(p * 128, 128)
            gathers.append(pltpu.async_copy(
                ci1_hbm.at[cj1_v.at[sl]], g11_v.at[sl], sem))
            gathers.append(pltpu.async_copy(
                ci2_hbm.at[cj1_v.at[sl]], g21_v.at[sl], sem))
            gathers.append(pltpu.async_copy(
                ci1_hbm.at[cj2_v.at[sl]], g12_v.at[sl], sem))
            gathers.append(pltpu.async_copy(
                ci2_hbm.at[cj2_v.at[sl]], g22_v.at[sl], sem))
        for cp in gathers:
            cp.wait()
        copies = []
        for c in range(nch):
            off = c * L
            lane = lax.iota(jnp.int32, L)
            e1 = er1_v[pl.ds(off, L)]
            e2 = er2_v[pl.ds(off, L)]
            sl = pl.ds(off, L)
            ivec = i_local + off + lane
            j1 = jnp.maximum(e1, 0)
            j2 = jnp.maximum(e2, 0)
            cond1 = ((e1 >= 0)
                     & ((g11_v[sl] == ivec) | (g21_v[sl] == ivec)))
            cond2 = ((e2 >= 0)
                     & ((g12_v[sl] == ivec) | (g22_v[sl] == ivec)))
            one = jnp.full((L,), 1, jnp.int32)
            zero = jnp.zeros((L,), jnp.int32)
            bit1 = jnp.where(cond1, one << ((j1 & 3) * 8), zero)
            bit2 = jnp.where(cond2, one << ((j2 & 3) * 8), zero)
            gb = (base_row + off + lane) * nw4
            w1 = gb + (j1 >> 2)
            w2 = gb + (j2 >> 2)
            sw = w1 == w2
            both = bit1 | bit2
            val_v[pl.ds(2 * c * L, L)] = jnp.where(sw, both, bit1)
            val_v[pl.ds((2 * c + 1) * L, L)] = jnp.where(sw, both, bit2)
            copies.append(pltpu.async_copy(
                val_v.at[pl.ds(2 * c * L, L)], kaw_out.at[w1], sem2))
            copies.append(pltpu.async_copy(
                val_v.at[pl.ds((2 * c + 1) * L, L)], kaw_out.at[w2], sem2))
        for cp in copies:
            cp.wait()

    kfun = pl.kernel(
        k,
        mesh=mesh,
        out_type=(),
        scratch_types=[
            pltpu.VMEM((rows_pw,), jnp.int32),
            pltpu.VMEM((rows_pw,), jnp.int32),
            pltpu.VMEM((rows_pw,), jnp.int32),
            pltpu.VMEM((rows_pw,), jnp.int32),
            pltpu.VMEM((rows_pw,), jnp.int32),
            pltpu.VMEM((rows_pw,), jnp.int32),
            pltpu.VMEM((rows_pw,), jnp.int32),
            pltpu.VMEM((rows_pw,), jnp.int32),
            pltpu.VMEM((2 * nch * L,), jnp.int32),
            pltpu.SemaphoreType.DMA,
            pltpu.SemaphoreType.DMA,
        ],
    )
    kaw_ref = jax.new_ref(kaw)
    kfun(er1, er2, ci1f, ci2f, kaw_ref)
    return kaw_ref[...]


def kernel(desc0, desc1, W, b_proj, w_match, b_match):
    b, m, d = desc0.shape
    n = desc1.shape[1]
    BM = 256
    nmb = m // BM
    grid = (b, 2 * nmb + 1)
    s = float(d) ** 0.25

    wt = W.T
    bp = b_proj.reshape(1, d)
    wm = w_match.reshape(d, 1)
    bm = b_match.reshape(1, 1)

    body = functools.partial(_body, BM=BM, n=n, d=d, nmb=nmb, s=s)

    f32 = jnp.float32
    i32 = jnp.int32
    out_shape = (
        jax.ShapeDtypeStruct((b, m, n), f32),
        jax.ShapeDtypeStruct((b, m + 1, n + 1), f32),
        jax.ShapeDtypeStruct((b, m, n // 4), i32),
        jax.ShapeDtypeStruct((b, 1, m), i32),
        jax.ShapeDtypeStruct((b, 1, m), i32),
        jax.ShapeDtypeStruct((b, 1, n), i32),
        jax.ShapeDtypeStruct((b, 1, n), i32),
    )
    kn, logscores, kaw, er1, er2, ci1, ci2 = pl.pallas_call(
        body,
        grid=grid,
        in_specs=[
            pl.BlockSpec((None, BM, d),
                         lambda bi, mi: (bi, jnp.minimum(mi, nmb - 1), 0)),
            pl.BlockSpec((None, n, d), lambda bi, mi: (bi, 0, 0)),
            pl.BlockSpec((d, d), lambda bi, mi: (0, 0)),
            pl.BlockSpec((1, d), lambda bi, mi: (0, 0)),
            pl.BlockSpec((d, 1), lambda bi, mi: (0, 0)),
            pl.BlockSpec((1, 1), lambda bi, mi: (0, 0)),
        ],
        out_specs=[
            pl.BlockSpec((None, BM, n),
                         lambda bi, mi: (bi, jnp.minimum(mi, nmb - 1), 0)),
            pl.BlockSpec((None, BM, n + 1),
                         lambda bi, mi: (bi, jnp.clip(mi - nmb, 0, nmb), 0)),
            pl.BlockSpec((None, BM, n // 4),
                         lambda bi, mi: (bi, jnp.clip(mi - nmb, 0, nmb - 1), 0)),
            pl.BlockSpec((None, 1, m), lambda bi, mi: (bi, 0, 0)),
            pl.BlockSpec((None, 1, m), lambda bi, mi: (bi, 0, 0)),
            pl.BlockSpec((None, 1, n), lambda bi, mi: (bi, 0, 0)),
            pl.BlockSpec((None, 1, n), lambda bi, mi: (bi, 0, 0)),
        ],
        out_shape=out_shape,
        scratch_shapes=[
            pltpu.VMEM((d, n), f32),
            pltpu.VMEM((m, n), f32),
            pltpu.VMEM((m, 1), f32),
            pltpu.VMEM((m, 1), f32),
            pltpu.VMEM((1, n), f32),
            pltpu.VMEM((1, n), f32),
            pltpu.VMEM((1, n), f32),
            pltpu.VMEM((1, n), f32),
            pltpu.VMEM((1, n), f32),
            pltpu.VMEM((m, 1), f32),
            pltpu.VMEM((m, 1), f32),
            pltpu.VMEM((m, 1), i32),
            pltpu.VMEM((m, 1), i32),
            pltpu.VMEM((1, n), f32),
            pltpu.VMEM((1, n), f32),
            pltpu.VMEM((1, n), i32),
            pltpu.VMEM((1, n), i32),
        ],
    )(desc0, desc1, wt, bp, wm, bm)

    patched = _sc_patch(er1.reshape(b * m), er2.reshape(b * m),
                        ci1.reshape(b * n), ci2.reshape(b * n),
                        kaw.reshape(b * m * (n // 4)), b, m, n)
    ka_bytes = lax.bitcast_convert_type(
        patched.reshape(b, m, n // 4), jnp.uint8)
    ka = ka_bytes.reshape(b, m, n).astype(jnp.bool_)
    return kn, logscores, ka


# speedup vs baseline: 1.5095x; 1.5095x over previous
import functools

import jax
import jax.numpy as jnp
from jax.experimental import pallas as pl
from jax.experimental.pallas import tpu as pltpu


def _body(d0_ref, d1_ref, wt_ref, bp_ref, wm_ref, bm_ref,
          kn_ref, ls_ref, ka_ref,
          md1t_s, S_s, ra_s, l0m_s, ls1_s, l1m_s,
          cmax_s, csum_s, cc_s, rv1_s, rv2_s, ri1_s, ri2_s,
          cv1_s, cv2_s, ci1_s, ci2_s,
          *, BM, n, d, nmb, s):
    mi = pl.program_id(1)
    NEG = jnp.float32(-jnp.inf)

    @pl.when(mi == 0)
    def _init():
        md1 = (jnp.dot(d1_ref[...], wt_ref[...],
                       preferred_element_type=jnp.float32) + bp_ref[...]) / s
        md1t_s[...] = md1.T
        z1 = jnp.dot(d1_ref[...], wm_ref[...],
                     preferred_element_type=jnp.float32) + bm_ref[...]
        ls1_s[...] = jax.nn.log_sigmoid(z1).T
        l1m_s[...] = jax.nn.log_sigmoid(-z1).T
        cmax_s[...] = jnp.full((1, n), NEG, jnp.float32)
        csum_s[...] = jnp.zeros((1, n), jnp.float32)

    @pl.when(mi < nmb)
    def _phase_a():
        d0 = d0_ref[...]
        md0 = (jnp.dot(d0, wt_ref[...],
                       preferred_element_type=jnp.float32) + bp_ref[...]) / s
        kn = jnp.dot(md0, md1t_s[...], preferred_element_type=jnp.float32)
        kn_ref[...] = kn
        S_s[pl.ds(mi * BM, BM), :] = kn
        rmax = jnp.max(kn, axis=1, keepdims=True)
        rlse = rmax + jnp.log(
            jnp.sum(jnp.exp(kn - rmax), axis=1, keepdims=True))
        z0 = jnp.dot(d0, wm_ref[...],
                     preferred_element_type=jnp.float32) + bm_ref[...]
        ra_s[pl.ds(mi * BM, BM), :] = jax.nn.log_sigmoid(z0) - rlse
        l0m_s[pl.ds(mi * BM, BM), :] = jax.nn.log_sigmoid(-z0)
        bmax = jnp.max(kn, axis=0, keepdims=True)
        prev = cmax_s[...]
        newm = jnp.maximum(prev, bmax)
        csum_s[...] = csum_s[...] * jnp.exp(prev - newm) + jnp.sum(
            jnp.exp(kn - newm), axis=0, keepdims=True)
        cmax_s[...] = newm

    @pl.when(mi == nmb)
    def _fold_col():
        cc_s[...] = ls1_s[...] - (cmax_s[...] + jnp.log(csum_s[...]))

    @pl.when((mi >= nmb) & (mi < 2 * nmb))
    def _phase_b():
        k = mi - nmb
        kn = S_s[pl.ds(k * BM, BM), :]
        core = (kn + kn) + ra_s[pl.ds(k * BM, BM), :] + cc_s[...]
        ls_ref[:, 0:n] = core
        ls_ref[:, n:n + 1] = l0m_s[pl.ds(k * BM, BM), :]
        jj = jax.lax.broadcasted_iota(jnp.int32, (BM, n), 1)
        v1 = jnp.max(core, axis=1, keepdims=True)
        i1 = jnp.min(jnp.where(core == v1, jj, n), axis=1, keepdims=True)
        sm = jnp.where(jj == i1, NEG, core)
        v2 = jnp.max(sm, axis=1, keepdims=True)
        i2 = jnp.min(jnp.where(sm == v2, jj, n), axis=1, keepdims=True)
        rv1_s[pl.ds(k * BM, BM), :] = v1
        rv2_s[pl.ds(k * BM, BM), :] = v2
        ri1_s[pl.ds(k * BM, BM), :] = i1
        ri2_s[pl.ds(k * BM, BM), :] = i2
        ii = jax.lax.broadcasted_iota(jnp.int32, (BM, n), 0)
        bv1 = jnp.max(core, axis=0, keepdims=True)
        bl1 = jnp.min(jnp.where(core == bv1, ii, BM), axis=0, keepdims=True)
        sc = jnp.where(ii == bl1, NEG, core)
        bv2 = jnp.max(sc, axis=0, keepdims=True)
        bl2 = jnp.min(jnp.where(sc == bv2, ii, BM), axis=0, keepdims=True)
        gbi1 = bl1 + k * BM
        gbi2 = bl2 + k * BM
        first = k == 0
        pv1 = jnp.where(first, NEG, cv1_s[...])
        pi1 = jnp.where(first, 0, ci1_s[...])
        pv2 = jnp.where(first, NEG, cv2_s[...])
        pi2 = jnp.where(first, 0, ci2_s[...])
        run1 = pv1 >= bv1
        cv1_s[...] = jnp.where(run1, pv1, bv1)
        ci1_s[...] = jnp.where(run1, pi1, gbi1)
        cv2_s[...] = jnp.where(run1, jnp.maximum(pv2, bv1),
                               jnp.maximum(pv1, bv2))
        ci2_s[...] = jnp.where(run1, jnp.where(pv2 >= bv1, pi2, gbi1),
                               jnp.where(pv1 >= bv2, pi1, gbi2))

    @pl.when(mi == 2 * nmb)
    def _border_row():
        ls_ref[0:1, 0:n] = l1m_s[...]
        ls_ref[0:1, n:n + 1] = jnp.zeros((1, 1), jnp.float32)
        ri1_s[...] = jnp.where(jnp.exp(rv1_s[...]) > 0.0, ri1_s[...], -1)
        ri2_s[...] = jnp.where(jnp.exp(rv2_s[...]) > 0.0, ri2_s[...], -1)

    @pl.when(mi >= 2 * nmb)
    def _phase_c():
        c = mi - 2 * nmb
        gi = c * BM + jax.lax.broadcasted_iota(jnp.int32, (BM, n), 0)
        jj = jax.lax.broadcasted_iota(jnp.int32, (BM, n), 1)
        ri1 = ri1_s[pl.ds(c * BM, BM), :]
        ri2 = ri2_s[pl.ds(c * BM, BM), :]
        rowm = (jj == ri1) | (jj == ri2)
        colm = (gi == ci1_s[...]) | (gi == ci2_s[...])
        ka_ref[...] = rowm & colm


def kernel(desc0, desc1, W, b_proj, w_match, b_match):
    b, m, d = desc0.shape
    n = desc1.shape[1]
    BM = 512
    nmb = m // BM
    grid = (b, 3 * nmb)
    s = float(d) ** 0.25

    wt = W.T
    bp = b_proj.reshape(1, d)
    wm = w_match.reshape(d, 1)
    bm = b_match.reshape(1, 1)

    body = functools.partial(_body, BM=BM, n=n, d=d, nmb=nmb, s=s)

    f32 = jnp.float32
    i32 = jnp.int32
    out_shape = (
        jax.ShapeDtypeStruct((b, m, n), f32),
        jax.ShapeDtypeStruct((b, m + 1, n + 1), f32),
        jax.ShapeDtypeStruct((b, m, n), jnp.bool_),
    )
    kn, logscores, ka = pl.pallas_call(
        body,
        grid=grid,
        in_specs=[
            pl.BlockSpec((None, BM, d),
                         lambda bi, mi: (bi, jnp.minimum(mi, nmb - 1), 0)),
            pl.BlockSpec((None, n, d), lambda bi, mi: (bi, 0, 0)),
            pl.BlockSpec((d, d), lambda bi, mi: (0, 0)),
            pl.BlockSpec((1, d), lambda bi, mi: (0, 0)),
            pl.BlockSpec((d, 1), lambda bi, mi: (0, 0)),
            pl.BlockSpec((1, 1), lambda bi, mi: (0, 0)),
        ],
        out_specs=[
            pl.BlockSpec((None, BM, n),
                         lambda bi, mi: (bi, jnp.minimum(mi, nmb - 1), 0)),
            pl.BlockSpec((None, BM, n + 1),
                         lambda bi, mi: (bi, jnp.clip(mi - nmb, 0, nmb), 0)),
            pl.BlockSpec((None, BM, n),
                         lambda bi, mi: (bi, jnp.clip(mi - 2 * nmb, 0, nmb - 1), 0)),
        ],
        out_shape=out_shape,
        scratch_shapes=[
            pltpu.VMEM((d, n), f32),
            pltpu.VMEM((m, n), f32),
            pltpu.VMEM((m, 1), f32),
            pltpu.VMEM((m, 1), f32),
            pltpu.VMEM((1, n), f32),
            pltpu.VMEM((1, n), f32),
            pltpu.VMEM((1, n), f32),
            pltpu.VMEM((1, n), f32),
            pltpu.VMEM((1, n), f32),
            pltpu.VMEM((m, 1), f32),
            pltpu.VMEM((m, 1), f32),
            pltpu.VMEM((m, 1), i32),
            pltpu.VMEM((m, 1), i32),
            pltpu.VMEM((1, n), f32),
            pltpu.VMEM((1, n), f32),
            pltpu.VMEM((1, n), i32),
            pltpu.VMEM((1, n), i32),
        ],
    )(desc0, desc1, wt, bp, wm, bm)
    return kn, logscores, ka
